# baseline (speedup 1.0000x reference)
import jax
import jax.numpy as jnp
from jax import lax
from jax.experimental import pallas as pl
from jax.experimental.pallas import tpu as pltpu
from jax.experimental.pallas import tpu_sc as plsc

N = 10000
D = 128
E = 320000
NC = 2
NS = 16
NW = NC * NS
K = 80
NCHUNK = 128
EPW = K * NCHUNK
NP = 10240
RPS = NP // NS
PRING = 4


def _sc_aggregate_kernel(x_hbm, adj_hbm, featp_hbm, degp_hbm,
                         p0_v, p1_v, p2_v, p3_v,
                         s0_v, s1_v, d0_v, d1_v,
                         rows_v, ones_v, accf_s, accd_s, gsem, psem):
  c = lax.axis_index("c")
  s = lax.axis_index("s")
  w = c * NS + s
  prefs = (p0_v, p1_v, p2_v, p3_v)
  srefs = (s0_v, s1_v)
  drefs = (d0_v, d1_v)

  def _init_ones(i, carry):
    ones_v[pl.ds(i * 16, 16)] = jnp.ones((16,), jnp.float32)
    return carry
  lax.fori_loop(0, K // 16, _init_ones, 0)

  def _zero_row(i, carry):
    for j in range(8):
      rows_v[0, i, pl.ds(j * 16, 16)] = jnp.zeros((16,), jnp.float32)
    return carry
  lax.fori_loop(0, K, _zero_row, 0)

  rowbase = s * RPS
  for t in range(RPS // K):
    pltpu.sync_copy(rows_v.at[0], accf_s.at[pl.ds(rowbase + t * K, K)])
  for t in range(RPS // 128):
    pltpu.sync_copy(rows_v.at[0, t], accd_s.at[pl.ds(rowbase + t * 128, 128)])
  plsc.subcore_barrier()

  pbase = w * EPW * 2

  def _pload(j, q):
    pltpu.async_copy(adj_hbm.at[pl.ds(pbase + j * 2 * K, 2 * K)],
                     prefs[q], psem.at[q])

  def _pwait(q):
    pltpu.make_async_copy(adj_hbm.at[pl.ds(pbase, 2 * K)], prefs[q],
                          psem.at[q]).wait()

  lane = jax.lax.iota(jnp.int32, 16)
  shuf = (lane % 8) * 2
  low = lane < 8
  _dn = jax.lax.GatherDimensionNumbers(
      offset_dims=(), collapsed_slice_dims=(0,), start_index_map=(0,))

  def _take(v, idx):
    return jax.lax.gather(
        v, idx.reshape(16, 1), dimension_numbers=_dn, slice_sizes=(1,),
        mode=jax.lax.GatherScatterMode.PROMISE_IN_BOUNDS)

  def _deint(q, h):
    for t in range(K // 16):
      a = prefs[q][pl.ds(t * 32, 16)]
      bb = prefs[q][pl.ds(t * 32 + 16, 16)]
      evens = jnp.where(low, _take(a, shuf), _take(bb, shuf))
      odds = jnp.where(low, _take(a, shuf + 1), _take(bb, shuf + 1))
      drefs[h][pl.ds(t * 16, 16)] = evens
      srefs[h][pl.ds(t * 16, 16)] = odds

  def _gather(h, b):
    pltpu.async_copy(x_hbm.at[srefs[h]], rows_v.at[b], gsem.at[b])

  for q in range(PRING):
    _pload(q, q)
  _pwait(0)
  _deint(0, 0)
  _gather(0, 0)

  def _step(g, carry):
    for q in range(PRING):
      i = g * PRING + q
      b = q % 2
      pltpu.make_async_copy(x_hbm.at[srefs[0]], rows_v.at[b],
                            gsem.at[b]).wait()

      qn = (q + 1) % PRING

      @pl.when(i + 1 < NCHUNK)
      def _():
        _pwait(qn)

      _deint(qn, 1 - b)

      @pl.when(i + 1 < NCHUNK)
      def _():
        _gather(1 - b, 1 - b)

      pltpu.sync_copy(rows_v.at[b], accf_s.at[drefs[b]], add=True)
      pltpu.sync_copy(ones_v, accd_s.at[drefs[b]], add=True)

      @pl.when(i + PRING < NCHUNK)
      def _():
        _pload(i + PRING, q)
    return carry
  lax.fori_loop(0, NCHUNK // PRING, _step, 0)

  plsc.subcore_barrier()

  pltpu.sync_copy(accf_s.at[pl.ds(rowbase, RPS)],
                  featp_hbm.at[c, pl.ds(rowbase, RPS)])
  pltpu.sync_copy(accd_s.at[pl.ds(rowbase, RPS)],
                  degp_hbm.at[pl.ds(c * NP + rowbase, RPS)])


@jax.jit
def _sc_aggregate(x, adj):
  mesh = plsc.VectorSubcoreMesh(core_axis_name="c", subcore_axis_name="s")
  return pl.kernel(
      _sc_aggregate_kernel,
      out_type=[
          jax.ShapeDtypeStruct((NC, NP, D), jnp.float32),
          jax.ShapeDtypeStruct((NC * NP,), jnp.float32),
      ],
      mesh=mesh,
      scratch_types=[
          pltpu.VMEM((2 * K,), jnp.int32),
          pltpu.VMEM((2 * K,), jnp.int32),
          pltpu.VMEM((2 * K,), jnp.int32),
          pltpu.VMEM((2 * K,), jnp.int32),
          pltpu.VMEM((K,), jnp.int32),
          pltpu.VMEM((K,), jnp.int32),
          pltpu.VMEM((K,), jnp.int32),
          pltpu.VMEM((K,), jnp.int32),
          pltpu.VMEM((2, K, D), jnp.float32),
          pltpu.VMEM((K,), jnp.float32),
          pltpu.VMEM_SHARED((NP, D), jnp.float32),
          pltpu.VMEM_SHARED((NP,), jnp.float32),
          pltpu.SemaphoreType.DMA((2,)),
          pltpu.SemaphoreType.DMA((PRING,)),
      ],
  )(x, adj)


BN = 1024


def _tc_finish_kernel(x_ref, fp_ref, dp_ref, w_ref, b_ref, o_ref):
  ssum = fp_ref[0] + fp_ref[1]
  deg = (dp_ref[0] + dp_ref[1]).reshape(BN, 1)
  h = ssum * (1.0 / deg)
  t = (jnp.dot(x_ref[...], w_ref[pl.ds(0, D)],
               preferred_element_type=jnp.float32)
       + jnp.dot(h, w_ref[pl.ds(D, D)],
                 preferred_element_type=jnp.float32)
       + b_ref[...])
  o_ref[...] = jax.nn.sigmoid(t)


@jax.jit
def _tc_finish(x, featp, degp, weight, bias):
  grid = ((N + BN - 1) // BN,)
  return pl.pallas_call(
      _tc_finish_kernel,
      grid=grid,
      in_specs=[
          pl.BlockSpec((BN, D), lambda i: (i, 0)),
          pl.BlockSpec((NC, BN, D), lambda i: (0, i, 0)),
          pl.BlockSpec((NC, BN), lambda i: (0, i)),
          pl.BlockSpec((2 * D, D), lambda i: (0, 0)),
          pl.BlockSpec((1, D), lambda i: (0, 0)),
      ],
      out_specs=pl.BlockSpec((BN, D), lambda i: (i, 0)),
      out_shape=jax.ShapeDtypeStruct((N, D), jnp.float32),
  )(x, featp, degp.reshape(NC, NP), weight, bias.reshape(1, D))


@jax.jit
def kernel(node_feat_input, adjacency_input, indices, weight, bias):
  del indices
  pad = NW * EPW - E
  adj = jnp.concatenate(
      [adjacency_input,
       jnp.tile(jnp.array([[N, 0]], jnp.int32), (pad, 1))],
      axis=0).reshape(NW * EPW * 2)
  featp, degp = _sc_aggregate(node_feat_input, adj)
  return _tc_finish(node_feat_input, featp, degp, weight, bias)


# speedup vs baseline: 1.2044x; 1.2044x over previous
import jax
import jax.numpy as jnp
from jax import lax
from jax.experimental import pallas as pl
from jax.experimental.pallas import tpu as pltpu
from jax.experimental.pallas import tpu_sc as plsc

N = 10000
D = 128
E = 320000
NC = 2
NS = 16
NW = NC * NS
K = 88
NCHUNK = 118
EPW = K * NCHUNK
NP = 10240
RPS = NP // NS
NBUF = 2


def _sc_aggregate_kernel(x_hbm, src_hbm, dst_hbm, featp_hbm, degp_hbm,
                         idxd_v, idxs_v, rows_v, ones_v,
                         accf_s, accd_s, gsem):
  c = lax.axis_index("c")
  s = lax.axis_index("s")
  w = c * NS + s

  def _init_ones(i, carry):
    ones_v[pl.ds(i * 16, 16)] = jnp.ones((16,), jnp.float32)
    return carry
  lax.fori_loop(0, 6, _init_ones, 0)

  def _zero_row(i, carry):
    for j in range(8):
      rows_v[0, i, pl.ds(j * 16, 16)] = jnp.zeros((16,), jnp.float32)
    return carry
  lax.fori_loop(0, K, _zero_row, 0)

  rowbase = s * RPS
  for t in range(RPS // 64):
    pltpu.sync_copy(rows_v.at[0, pl.ds(0, 64)],
                    accf_s.at[pl.ds(rowbase + t * 64, 64)])
  for t in range(RPS // 128):
    pltpu.sync_copy(rows_v.at[0, t], accd_s.at[pl.ds(rowbase + t * 128, 128)])

  pltpu.sync_copy(dst_hbm.at[w], idxd_v)
  pltpu.sync_copy(src_hbm.at[pl.ds(w * EPW, EPW)], idxs_v)
  plsc.subcore_barrier()

  def _gather(i, b):
    pltpu.async_copy(x_hbm.at[idxs_v.at[pl.ds(i * K, K)]], rows_v.at[b],
                     gsem.at[b])

  _gather(0, 0)

  def _step(g, carry):
    for b in range(NBUF):
      i = g * NBUF + b
      pltpu.make_async_copy(x_hbm.at[idxs_v.at[pl.ds(0, K)]], rows_v.at[b],
                            gsem.at[b]).wait()

      @pl.when(i + 1 < NCHUNK)
      def _():
        _gather(i + 1, 1 - b)

      pltpu.sync_copy(rows_v.at[b], accf_s.at[idxd_v.at[i]], add=True)
      pltpu.sync_copy(ones_v.at[pl.ds(0, K)], accd_s.at[idxd_v.at[i]], add=True)
    return carry
  lax.fori_loop(0, NCHUNK // NBUF, _step, 0)

  plsc.subcore_barrier()

  pltpu.sync_copy(accf_s.at[pl.ds(rowbase, RPS)],
                  featp_hbm.at[c, pl.ds(rowbase, RPS)])
  pltpu.sync_copy(accd_s.at[pl.ds(rowbase, RPS)],
                  degp_hbm.at[pl.ds(c * NP + rowbase, RPS)])


@jax.jit
def _sc_aggregate(x, src, dst):
  mesh = plsc.VectorSubcoreMesh(core_axis_name="c", subcore_axis_name="s")
  return pl.kernel(
      _sc_aggregate_kernel,
      out_type=[
          jax.ShapeDtypeStruct((NC, NP, D), jnp.float32),
          jax.ShapeDtypeStruct((NC * NP,), jnp.float32),
      ],
      mesh=mesh,
      scratch_types=[
          pltpu.VMEM((NCHUNK, K), jnp.int32),
          pltpu.VMEM((EPW,), jnp.int32),
          pltpu.VMEM((NBUF, K, D), jnp.float32),
          pltpu.VMEM((96,), jnp.float32),
          pltpu.VMEM_SHARED((NP, D), jnp.float32),
          pltpu.VMEM_SHARED((NP,), jnp.float32),
          pltpu.SemaphoreType.DMA((NBUF,)),
      ],
  )(x, src, dst)


BN = 1024


def _tc_finish_kernel(x_ref, fp_ref, dp_ref, w_ref, b_ref, o_ref):
  ssum = fp_ref[0] + fp_ref[1]
  deg = (dp_ref[0] + dp_ref[1]).reshape(BN, 1)
  h = ssum * (1.0 / deg)
  t = (jnp.dot(x_ref[...], w_ref[pl.ds(0, D)],
               preferred_element_type=jnp.float32)
       + jnp.dot(h, w_ref[pl.ds(D, D)],
                 preferred_element_type=jnp.float32)
       + b_ref[...])
  o_ref[...] = jax.nn.sigmoid(t)


@jax.jit
def _tc_finish(x, featp, degp, weight, bias):
  grid = ((N + BN - 1) // BN,)
  return pl.pallas_call(
      _tc_finish_kernel,
      grid=grid,
      in_specs=[
          pl.BlockSpec((BN, D), lambda i: (i, 0)),
          pl.BlockSpec((NC, BN, D), lambda i: (0, i, 0)),
          pl.BlockSpec((NC, BN), lambda i: (0, i)),
          pl.BlockSpec((2 * D, D), lambda i: (0, 0)),
          pl.BlockSpec((1, D), lambda i: (0, 0)),
      ],
      out_specs=pl.BlockSpec((BN, D), lambda i: (i, 0)),
      out_shape=jax.ShapeDtypeStruct((N, D), jnp.float32),
  )(x, featp, degp.reshape(NC, NP), weight, bias.reshape(1, D))


@jax.jit
def kernel(node_feat_input, adjacency_input, indices, weight, bias):
  del indices
  dst = adjacency_input[:, 0]
  src = adjacency_input[:, 1]
  pad = EPW - E // NW
  src3 = jnp.concatenate(
      [src.reshape(NW, E // NW),
       jnp.zeros((NW, pad), jnp.int32)], axis=1).reshape(NW * EPW)
  dst3 = jnp.concatenate(
      [dst.reshape(NW, E // NW),
       jnp.full((NW, pad), N, jnp.int32)], axis=1).reshape(NW, NCHUNK, K)
  featp, degp = _sc_aggregate(node_feat_input, src3, dst3)
  return _tc_finish(node_feat_input, featp, degp, weight, bias)


# speedup vs baseline: 2.6950x; 2.2376x over previous
import jax
import jax.numpy as jnp
from jax import lax
from jax.experimental import pallas as pl
from jax.experimental.pallas import tpu as pltpu
from jax.experimental.pallas import tpu_sc as plsc

N = 10000
D = 128
E = 320000
NC = 2
NS = 16
NW = NC * NS
K = 80
NCHUNK = 126
EPW = K * NCHUNK
NP = 10240
RPS = NP // NS
NBUF = 2


def _sc_aggregate_kernel(x_hbm, src_hbm, dst_hbm, featp_hbm, degp_hbm,
                         idxd_v, idxs_v, rows_v, ones_v,
                         accf_s, accd_s, gsem):
  c = lax.axis_index("c")
  s = lax.axis_index("s")
  w = c * NS + s

  def _init_ones(i, carry):
    ones_v[pl.ds(i * 16, 16)] = jnp.ones((16,), jnp.float32)
    return carry
  lax.fori_loop(0, K // 16, _init_ones, 0)

  def _zero_row(i, carry):
    for j in range(8):
      rows_v[0, i, pl.ds(j * 16, 16)] = jnp.zeros((16,), jnp.float32)
    return carry
  lax.fori_loop(0, K, _zero_row, 0)

  rowbase = s * RPS
  for t in range(RPS // K):
    pltpu.sync_copy(rows_v.at[0], accf_s.at[pl.ds(rowbase + t * K, K)])
  for t in range(RPS // 128):
    pltpu.sync_copy(rows_v.at[0, t], accd_s.at[pl.ds(rowbase + t * 128, 128)])

  pltpu.sync_copy(dst_hbm.at[w], idxd_v)
  pltpu.sync_copy(src_hbm.at[pl.ds(w * EPW, EPW)], idxs_v)
  plsc.subcore_barrier()

  def _gather(i, b):
    pltpu.async_copy(x_hbm.at[idxs_v.at[pl.ds(i * K, K)]], rows_v.at[b],
                     gsem.at[b])

  _gather(0, 0)

  def _step(g, carry):
    for b in range(NBUF):
      i = g * NBUF + b
      pltpu.make_async_copy(x_hbm.at[idxs_v.at[pl.ds(0, K)]], rows_v.at[b],
                            gsem.at[b]).wait()

      @pl.when(i + 1 < NCHUNK)
      def _():
        _gather(i + 1, 1 - b)

      pltpu.sync_copy(rows_v.at[b], accf_s.at[idxd_v.at[i]], add=True)
      pltpu.sync_copy(ones_v, accd_s.at[idxd_v.at[i]], add=True)
    return carry
  lax.fori_loop(0, NCHUNK // NBUF, _step, 0)

  plsc.subcore_barrier()

  pltpu.sync_copy(accf_s.at[pl.ds(rowbase, RPS)],
                  featp_hbm.at[c, pl.ds(rowbase, RPS)])
  pltpu.sync_copy(accd_s.at[pl.ds(rowbase, RPS)],
                  degp_hbm.at[pl.ds(c * NP + rowbase, RPS)])


@jax.jit
def _sc_aggregate(x, src, dst):
  mesh = plsc.VectorSubcoreMesh(core_axis_name="c", subcore_axis_name="s")
  return pl.kernel(
      _sc_aggregate_kernel,
      out_type=[
          jax.ShapeDtypeStruct((NC, NP, D), jnp.float32),
          jax.ShapeDtypeStruct((NC * NP,), jnp.float32),
      ],
      mesh=mesh,
      scratch_types=[
          pltpu.VMEM((NCHUNK, K), jnp.int32),
          pltpu.VMEM((EPW,), jnp.int32),
          pltpu.VMEM((NBUF, K, D), jnp.float32),
          pltpu.VMEM((K,), jnp.float32),
          pltpu.VMEM_SHARED((NP, D), jnp.float32),
          pltpu.VMEM_SHARED((NP,), jnp.float32),
          pltpu.SemaphoreType.DMA((NBUF,)),
      ],
  )(x, src, dst)


BN = 1024


def _tc_finish_kernel(x_ref, fp_ref, dp_ref, w_ref, b_ref, o_ref):
  ssum = fp_ref[0] + fp_ref[1]
  deg = (dp_ref[0] + dp_ref[1]).reshape(BN, 1)
  h = ssum * (1.0 / deg)
  t = (jnp.dot(x_ref[...], w_ref[pl.ds(0, D)],
               preferred_element_type=jnp.float32)
       + jnp.dot(h, w_ref[pl.ds(D, D)],
                 preferred_element_type=jnp.float32)
       + b_ref[...])
  o_ref[...] = jax.nn.sigmoid(t)


@jax.jit
def _tc_finish(x, featp, degp, weight, bias):
  grid = ((N + BN - 1) // BN,)
  return pl.pallas_call(
      _tc_finish_kernel,
      grid=grid,
      in_specs=[
          pl.BlockSpec((BN, D), lambda i: (i, 0)),
          pl.BlockSpec((NC, BN, D), lambda i: (0, i, 0)),
          pl.BlockSpec((NC, BN), lambda i: (0, i)),
          pl.BlockSpec((2 * D, D), lambda i: (0, 0)),
          pl.BlockSpec((1, D), lambda i: (0, 0)),
      ],
      out_specs=pl.BlockSpec((BN, D), lambda i: (i, 0)),
      out_shape=jax.ShapeDtypeStruct((N, D), jnp.float32),
  )(x, featp, degp.reshape(NC, NP), weight, bias.reshape(1, D))


@jax.jit
def kernel(node_feat_input, adjacency_input, indices, weight, bias):
  del indices
  dst = adjacency_input[:, 0]
  src = adjacency_input[:, 1]
  pad = EPW - E // NW
  src3 = jnp.concatenate(
      [src.reshape(NW, E // NW),
       jnp.zeros((NW, pad), jnp.int32)], axis=1).reshape(NW * EPW)
  dst3 = jnp.concatenate(
      [dst.reshape(NW, E // NW),
       jnp.full((NW, pad), N, jnp.int32)], axis=1).reshape(NW, NCHUNK, K)
  featp, degp = _sc_aggregate(node_feat_input, src3, dst3)
  return _tc_finish(node_feat_input, featp, degp, weight, bias)
